# Initial kernel scaffold; baseline (speedup 1.0000x reference)
import jax
import jax.numpy as jnp
from jax import lax
from jax.experimental import pallas as pl
from jax.experimental.pallas import tpu as pltpu
from jax.experimental.pallas import tpu_sc as plsc

B, D = 128, 32768
L = 16
NC, NS = 2, 16
NW = NC * NS
ROWS_PER_W = B // NW
CHUNKS = D // L
UNROLL = 8
STEPS = CHUNKS // UNROLL
N_ITERS = 30


def _body(x_hbm, out_hbm, row_v):
    c = lax.axis_index("c")
    s = lax.axis_index("s")
    wid = s * NC + c

    def do_row(r, _):
        row = wid * ROWS_PER_W + r
        pltpu.sync_copy(x_hbm.at[row], row_v)

        def amax_body(i, m):
            base = i * (UNROLL * L)
            for j in range(UNROLL):
                m = jnp.maximum(m, row_v[pl.ds(base + j * L, L)])
            return m
        m = lax.fori_loop(0, STEPS, amax_body,
                          jnp.full((L,), -jnp.inf, jnp.float32))
        m_s = jnp.max(m)

        def exp_body(i, accs):
            a0, a1 = accs
            base = i * (UNROLL * L)
            for j in range(UNROLL):
                u = jnp.exp(row_v[pl.ds(base + j * L, L)] - m_s)
                row_v[pl.ds(base + j * L, L)] = u
                if j % 2 == 0:
                    a0 = a0 + u
                else:
                    a1 = a1 + u
            return a0, a1
        z0, z1 = lax.fori_loop(0, STEPS, exp_body,
                               (jnp.zeros((L,), jnp.float32),
                                jnp.zeros((L,), jnp.float32)))
        z_s = jnp.sum(z0 + z1)

        def div_body(i, carry):
            a0, a1, pm = carry
            base = i * (UNROLL * L)
            for j in range(UNROLL):
                p = row_v[pl.ds(base + j * L, L)] / z_s
                row_v[pl.ds(base + j * L, L)] = p
                if j % 2 == 0:
                    a0 = a0 + p
                else:
                    a1 = a1 + p
                pm = jnp.maximum(pm, p)
            return a0, a1, pm
        t0, t1, pmacc = lax.fori_loop(
            0, STEPS, div_body,
            (jnp.zeros((L,), jnp.float32), jnp.zeros((L,), jnp.float32),
             jnp.zeros((L,), jnp.float32)))
        total = jnp.sum(t0 + t1)
        pmax = jnp.max(pmacc)
        t_thresh = total - jnp.float32(0.5)

        def bs_body(_, carry):
            lo, hi, kept = carry
            mid = lo + lax.shift_right_logical(hi - lo, 1)
            t = lax.bitcast_convert_type(mid, jnp.float32)

            def w_body(i, accs):
                a0, a1, a2, a3 = accs
                base = i * (UNROLL * L)
                for j in range(UNROLL):
                    v = row_v[pl.ds(base + j * L, L)]
                    w = jnp.where(v > t, v, jnp.float32(0.0))
                    if j % 4 == 0:
                        a0 = a0 + w
                    elif j % 4 == 1:
                        a1 = a1 + w
                    elif j % 4 == 2:
                        a2 = a2 + w
                    else:
                        a3 = a3 + w
                return a0, a1, a2, a3
            zero = jnp.zeros((L,), jnp.float32)
            w0, w1, w2, w3 = lax.fori_loop(0, STEPS, w_body,
                                           (zero, zero, zero, zero))
            W = jnp.sum((w0 + w1) + (w2 + w3))
            pred = W > t_thresh
            lo = jnp.where(pred, mid, lo)
            hi = jnp.where(pred, hi, mid)
            kept = jnp.where(pred, W, kept)
            return lo, hi, kept

        lo0 = jnp.int32(0)
        hi0 = lax.bitcast_convert_type(pmax, jnp.int32)
        lo, hi, kept = lax.fori_loop(0, N_ITERS, bs_body, (lo0, hi0, total))
        t_lo = lax.bitcast_convert_type(lo, jnp.float32)
        inv = jnp.float32(1.0) / (kept + jnp.float32(1e-7))

        def out_body(i, _unused):
            base = i * (UNROLL * L)
            for j in range(UNROLL):
                p = row_v[pl.ds(base + j * L, L)]
                row_v[pl.ds(base + j * L, L)] = jnp.where(
                    p > t_lo, p * inv, jnp.float32(0.0))
            return 0
        lax.fori_loop(0, STEPS, out_body, 0)

        pltpu.sync_copy(row_v, out_hbm.at[row])
        return 0

    lax.fori_loop(0, ROWS_PER_W, do_row, 0)


@jax.jit
def kernel(logits):
    return pl.kernel(
        _body,
        out_type=jax.ShapeDtypeStruct((B, D), jnp.float32),
        mesh=plsc.VectorSubcoreMesh(core_axis_name="c", subcore_axis_name="s"),
        scratch_types=[pltpu.VMEM((D,), jnp.float32)],
    )(logits)


# speedup vs baseline: 87.3473x; 87.3473x over previous
import jax
import jax.numpy as jnp
from jax import lax
from jax.experimental import pallas as pl
from jax.experimental.pallas import tpu as pltpu
from jax.experimental.pallas import tpu_sc as plsc

B, D = 128, 32768
L = 16
NC, NS = 2, 16
NW = NC * NS
ROWS_PER_W = B // NW
CHUNKS = D // L
UNROLL = 8
STEPS = CHUNKS // UNROLL
N_ITERS = 30


def _body(x_hbm, out_hbm, row_v):
    c = lax.axis_index("c")
    s = lax.axis_index("s")
    wid = s * NC + c

    def do_row(r, _):
        row = wid * ROWS_PER_W + r
        pltpu.sync_copy(x_hbm.at[row], row_v)

        def amax_body(i, m):
            base = i * (UNROLL * L)
            for j in range(UNROLL):
                m = jnp.maximum(m, row_v[pl.ds(base + j * L, L)])
            return m
        m = lax.fori_loop(0, STEPS, amax_body,
                          jnp.full((L,), -jnp.inf, jnp.float32))
        m_s = jnp.max(m)

        def exp_body(i, accs):
            a0, a1 = accs
            base = i * (UNROLL * L)
            for j in range(UNROLL):
                u = jnp.exp(row_v[pl.ds(base + j * L, L)] - m_s)
                row_v[pl.ds(base + j * L, L)] = u
                if j % 2 == 0:
                    a0 = a0 + u
                else:
                    a1 = a1 + u
            return a0, a1
        z0, z1 = lax.fori_loop(0, STEPS, exp_body,
                               (jnp.zeros((L,), jnp.float32),
                                jnp.zeros((L,), jnp.float32)))
        z_s = jnp.sum(z0 + z1)

        def div_body(i, carry):
            a0, a1, pm = carry
            base = i * (UNROLL * L)
            for j in range(UNROLL):
                p = row_v[pl.ds(base + j * L, L)] / z_s
                row_v[pl.ds(base + j * L, L)] = p
                if j % 2 == 0:
                    a0 = a0 + p
                else:
                    a1 = a1 + p
                pm = jnp.maximum(pm, p)
            return a0, a1, pm
        t0, t1, pmacc = lax.fori_loop(
            0, STEPS, div_body,
            (jnp.zeros((L,), jnp.float32), jnp.zeros((L,), jnp.float32),
             jnp.zeros((L,), jnp.float32)))
        total = jnp.sum(t0 + t1)
        pmax = jnp.max(pmacc)
        t_thresh = total - jnp.float32(0.5)

        def bs_body(_, carry):
            lo, hi, kept = carry
            mid = lo + lax.shift_right_logical(hi - lo, 1)
            t = lax.bitcast_convert_type(mid, jnp.float32)

            def w_body(i, accs):
                a0, a1, a2, a3 = accs
                base = i * (UNROLL * L)
                for j in range(UNROLL):
                    v = row_v[pl.ds(base + j * L, L)]
                    w = jnp.where(v > t, v, jnp.float32(0.0))
                    if j % 4 == 0:
                        a0 = a0 + w
                    elif j % 4 == 1:
                        a1 = a1 + w
                    elif j % 4 == 2:
                        a2 = a2 + w
                    else:
                        a3 = a3 + w
                return a0, a1, a2, a3
            zero = jnp.zeros((L,), jnp.float32)
            w0, w1, w2, w3 = lax.fori_loop(0, STEPS, w_body,
                                           (zero, zero, zero, zero))
            W = jnp.sum((w0 + w1) + (w2 + w3))
            pred = W > t_thresh
            lo = jnp.where(pred, mid, lo)
            hi = jnp.where(pred, hi, mid)
            kept = jnp.where(pred, W, kept)
            return lo, hi, kept

        lo0 = jnp.int32(0)
        hi0 = lax.bitcast_convert_type(pmax, jnp.int32)
        lo, hi, kept = lax.fori_loop(0, N_ITERS, bs_body, (lo0, hi0, total))
        t_lo = lax.bitcast_convert_type(lo, jnp.float32)
        inv = jnp.full((L,), 1.0, jnp.float32) / (kept + jnp.float32(1e-7))

        def out_body(i, _unused):
            base = i * (UNROLL * L)
            for j in range(UNROLL):
                p = row_v[pl.ds(base + j * L, L)]
                row_v[pl.ds(base + j * L, L)] = jnp.where(
                    p > t_lo, p * inv, jnp.float32(0.0))
            return 0
        lax.fori_loop(0, STEPS, out_body, 0)

        pltpu.sync_copy(row_v, out_hbm.at[row])
        return 0

    lax.fori_loop(0, ROWS_PER_W, do_row, 0)


@jax.jit
def kernel(logits):
    return pl.kernel(
        _body,
        out_type=jax.ShapeDtypeStruct((B, D), jnp.float32),
        mesh=plsc.VectorSubcoreMesh(core_axis_name="c", subcore_axis_name="s"),
        scratch_types=[pltpu.VMEM((D,), jnp.float32)],
        compiler_params=pltpu.CompilerParams(needs_layout_passes=False),
    )(logits)
